# initial kernel scaffold (unmeasured)
import jax
import jax.numpy as jnp
from jax import lax
from jax.experimental import pallas as pl
from jax.experimental.pallas import tpu as pltpu

N_DEV = 4
S = 1024
D = 1024
HQ_PER = 8
DH = 128
SCALE = 0.08838834764831843
NEG = -1e9


def kernel(x, Wq, K_ext, V_ext, Wo):
    my = lax.axis_index("i")

    xb = x.reshape(S, D).astype(jnp.bfloat16)
    wqwo = jnp.concatenate(
        [Wq.astype(jnp.bfloat16), Wo.astype(jnp.bfloat16)], axis=0
    )

    order = (my - jnp.arange(N_DEV)) % N_DEV
    Km = lax.dynamic_index_in_dim(K_ext, my, 0, keepdims=False)
    Vm = lax.dynamic_index_in_dim(V_ext, my, 0, keepdims=False)
    Kb = jnp.take(Km.reshape(S, N_DEV, D), order, axis=1).reshape(S, N_DEV * D)
    Vb = jnp.take(Vm.reshape(S, N_DEV, D), order, axis=1).reshape(S, N_DEV * D)
    Kb = Kb.astype(jnp.bfloat16)
    Vb = Vb.astype(jnp.bfloat16)

    def body(x_ref, w_ref, k_ref, v_ref, out_ref, comm_ref, send_sems, recv_sems):
        my_pos = lax.axis_index("i")
        right = lax.rem(my_pos + 1, N_DEV)
        left = lax.rem(my_pos + N_DEV - 1, N_DEV)

        barrier = pltpu.get_barrier_semaphore()
        for nbr in (left, right):
            pl.semaphore_signal(
                barrier, inc=1, device_id=(nbr,),
                device_id_type=pl.DeviceIdType.MESH,
            )
        pl.semaphore_wait(barrier, 2)

        qb = lax.broadcasted_iota(jnp.int32, (S, S), 0) // 64
        kb = lax.broadcasted_iota(jnp.int32, (S, S), 1) // 64
        mask = (qb == kb) | (kb == 0) | ((qb + kb) % 3 == 0)

        xv = x_ref[...]

        def compute_block(t):
            w = w_ref[...] if t == 0 else comm_ref[t - 1]
            wq = w[:D, :]
            wo = w[D:, :]
            q = jnp.dot(xv, wq, preferred_element_type=jnp.float32)
            q = q.astype(jnp.bfloat16)
            ctx_cols = []
            for h in range(HQ_PER):
                qh = q[:, h * DH:(h + 1) * DH]
                kh = k_ref[:, t * D + h * DH: t * D + (h + 1) * DH]
                s = lax.dot_general(
                    qh, kh, (((1,), (1,)), ((), ())),
                    preferred_element_type=jnp.float32,
                ) * SCALE
                s = jnp.where(mask, s, NEG)
                m = jnp.max(s, axis=1, keepdims=True)
                e = jnp.exp(s - m)
                p = (e / jnp.sum(e, axis=1, keepdims=True)).astype(jnp.bfloat16)
                vh = v_ref[:, t * D + h * DH: t * D + (h + 1) * DH]
                ctx_cols.append(
                    jnp.dot(p, vh, preferred_element_type=jnp.float32)
                    .astype(jnp.bfloat16)
                )
            ctx = jnp.concatenate(ctx_cols, axis=1)
            return jnp.dot(ctx, wo, preferred_element_type=jnp.float32)

        for h in range(N_DEV - 1):
            src = w_ref if h == 0 else comm_ref.at[h - 1]
            rdma = pltpu.make_async_remote_copy(
                src_ref=src,
                dst_ref=comm_ref.at[h],
                send_sem=send_sems.at[h],
                recv_sem=recv_sems.at[h],
                device_id=(right,),
                device_id_type=pl.DeviceIdType.MESH,
            )
            rdma.start()
            contrib = compute_block(h)
            if h == 0:
                out_ref[...] = contrib
            else:
                out_ref[...] += contrib
            rdma.wait()
        out_ref[...] += compute_block(N_DEV - 1)

    out = pl.pallas_call(
        body,
        out_shape=jax.ShapeDtypeStruct((S, D), jnp.float32),
        in_specs=[pl.BlockSpec(memory_space=pltpu.VMEM)] * 4,
        out_specs=pl.BlockSpec(memory_space=pltpu.VMEM),
        scratch_shapes=[
            pltpu.VMEM((N_DEV - 1, 2 * D, D), jnp.bfloat16),
            pltpu.SemaphoreType.DMA((N_DEV - 1,)),
            pltpu.SemaphoreType.DMA((N_DEV - 1,)),
        ],
        compiler_params=pltpu.CompilerParams(collective_id=0),
    )(xb, wqwo, Kb, Vb)
    return out.reshape(1, S, D)


# baseline (device time: 252800 ns/iter reference)
import jax
import jax.numpy as jnp
from jax import lax
from jax.experimental import pallas as pl
from jax.experimental.pallas import tpu as pltpu

N_DEV = 4
S = 1024
D = 1024
HQ_PER = 8
DH = 128
SCALE = 0.08838834764831843
NEG = -1e9


def kernel(x, Wq, K_ext, V_ext, Wo):
    my = lax.axis_index("i")

    xb = x.reshape(S, D).astype(jnp.bfloat16)
    wqwo = jnp.concatenate(
        [Wq.astype(jnp.bfloat16), Wo.astype(jnp.bfloat16)], axis=0
    )

    order = (my - jnp.arange(N_DEV)) % N_DEV
    Km = lax.dynamic_index_in_dim(K_ext, my, 0, keepdims=False)
    Vm = lax.dynamic_index_in_dim(V_ext, my, 0, keepdims=False)
    Kb = jnp.take(Km.reshape(S, N_DEV, D), order, axis=1).reshape(S, N_DEV * D)
    Vb = jnp.take(Vm.reshape(S, N_DEV, D), order, axis=1).reshape(S, N_DEV * D)
    Kb = Kb.astype(jnp.bfloat16)
    Vb = Vb.astype(jnp.bfloat16)

    def body(x_ref, w_ref, k_ref, v_ref, out_ref, comm_ref, send_sems, recv_sems):
        my_pos = lax.axis_index("i")
        right = lax.rem(my_pos + 1, N_DEV)
        left = lax.rem(my_pos + N_DEV - 1, N_DEV)

        barrier = pltpu.get_barrier_semaphore()
        for nbr in (left, right):
            pl.semaphore_signal(
                barrier, inc=1, device_id=(nbr,),
                device_id_type=pl.DeviceIdType.MESH,
            )
        pl.semaphore_wait(barrier, 2)

        qb = lax.broadcasted_iota(jnp.int32, (S, S), 0) // 64
        kb = lax.broadcasted_iota(jnp.int32, (S, S), 1) // 64
        mask = (qb == kb) | (kb == 0) | ((qb + kb) % 3 == 0)

        xv = x_ref[...]

        def compute_block(t):
            w = w_ref[...] if t == 0 else comm_ref[t - 1]
            wq = w[:D, :]
            wo = w[D:, :]
            q = jnp.dot(xv, wq, preferred_element_type=jnp.float32)
            q = q.astype(jnp.bfloat16)
            ctx_cols = []
            for h in range(HQ_PER):
                qh = q[:, h * DH:(h + 1) * DH]
                kh = k_ref[:, t * D + h * DH: t * D + (h + 1) * DH]
                s = lax.dot_general(
                    qh, kh, (((1,), (1,)), ((), ())),
                    preferred_element_type=jnp.float32,
                ) * SCALE
                s = jnp.where(mask, s, NEG)
                m = jnp.max(s, axis=1, keepdims=True)
                e = jnp.exp(s - m)
                p = (e / jnp.sum(e, axis=1, keepdims=True)).astype(jnp.bfloat16)
                vh = v_ref[:, t * D + h * DH: t * D + (h + 1) * DH]
                ctx_cols.append(
                    jnp.dot(p, vh, preferred_element_type=jnp.float32)
                    .astype(jnp.bfloat16)
                )
            ctx = jnp.concatenate(ctx_cols, axis=1)
            return jnp.dot(ctx, wo, preferred_element_type=jnp.float32)

        for h in range(N_DEV - 1):
            src = w_ref if h == 0 else comm_ref.at[h - 1]
            rdma = pltpu.make_async_remote_copy(
                src_ref=src,
                dst_ref=comm_ref.at[h],
                send_sem=send_sems.at[h],
                recv_sem=recv_sems.at[h],
                device_id=(right,),
                device_id_type=pl.DeviceIdType.MESH,
            )
            rdma.start()
            contrib = compute_block(h)
            if h == 0:
                out_ref[...] = contrib
            else:
                out_ref[...] += contrib
            rdma.wait()
        out_ref[...] += compute_block(N_DEV - 1)

    out = pl.pallas_call(
        body,
        out_shape=jax.ShapeDtypeStruct((S, D), jnp.float32),
        in_specs=[pl.BlockSpec(memory_space=pltpu.VMEM)] * 4,
        out_specs=pl.BlockSpec(memory_space=pltpu.VMEM),
        scratch_shapes=[
            pltpu.VMEM((N_DEV - 1, 2 * D, D), jnp.bfloat16),
            pltpu.SemaphoreType.DMA((N_DEV - 1,)),
            pltpu.SemaphoreType.DMA((N_DEV - 1,)),
        ],
        compiler_params=pltpu.CompilerParams(
            collective_id=0, vmem_limit_bytes=56 * 1024 * 1024
        ),
    )(xb, wqwo, Kb, Vb)
    return out.reshape(1, S, D)


# device time: 205638 ns/iter; 1.2293x vs baseline; 1.2293x over previous
import jax
import jax.numpy as jnp
from jax import lax
from jax.experimental import pallas as pl
from jax.experimental.pallas import tpu as pltpu

N_DEV = 4
S = 1024
D = 1024
HH = 4
DH = 128
DHALF = HH * DH
SCALE = 0.08838834764831843
NEG = -1e9


def kernel(x, Wq, K_ext, V_ext, Wo):
    my = lax.axis_index("i")

    xb = x.reshape(S, D).astype(jnp.bfloat16)
    Wq16 = Wq.astype(jnp.bfloat16)
    Wo16 = Wo.astype(jnp.bfloat16)
    wR = jnp.concatenate([Wq16[:, :DHALF], Wo16[:DHALF, :].T], axis=0)
    wL = jnp.concatenate([Wq16[:, DHALF:], Wo16[DHALF:, :].T], axis=0)

    order_r = (my - jnp.arange(N_DEV)) % N_DEV
    order_l = (my + jnp.arange(N_DEV)) % N_DEV
    Km = lax.dynamic_index_in_dim(K_ext, my, 0, keepdims=False)
    Vm = lax.dynamic_index_in_dim(V_ext, my, 0, keepdims=False)

    def prep(A):
        A4 = A.reshape(S, N_DEV, D)
        right = jnp.take(A4, order_r, axis=1)[:, :, :DHALF]
        left = jnp.take(A4, order_l, axis=1)[:, :, DHALF:]
        out = jnp.concatenate(
            [right.reshape(S, N_DEV * DHALF), left.reshape(S, N_DEV * DHALF)],
            axis=1,
        )
        return out.astype(jnp.bfloat16)

    Kb = prep(Km)
    Vb = prep(Vm)

    def body(x_ref, wr_ref, wl_ref, k_ref, v_ref, out_ref,
             slotr, slotl, sr_send, sr_recv, sl_send, sl_recv):
        my_pos = lax.axis_index("i")
        right = lax.rem(my_pos + 1, N_DEV)
        left = lax.rem(my_pos + N_DEV - 1, N_DEV)

        barrier = pltpu.get_barrier_semaphore()
        for nbr in (left, right):
            pl.semaphore_signal(
                barrier, inc=1, device_id=(nbr,),
                device_id_type=pl.DeviceIdType.MESH,
            )
        pl.semaphore_wait(barrier, 2)

        qb = lax.broadcasted_iota(jnp.int32, (S, S), 0) // 64
        kb = lax.broadcasted_iota(jnp.int32, (S, S), 1) // 64
        mask = (qb == kb) | (kb == 0) | ((qb + kb) % 3 == 0)
        bias = jnp.where(mask, 0.0, NEG).astype(jnp.float32)

        xv = x_ref[...]

        def compute_half(w, col0):
            wq = w[:D, :]
            woT = w[D:, :]
            q = (jnp.dot(xv, wq, preferred_element_type=jnp.float32)
                 * SCALE).astype(jnp.bfloat16)
            ctx_cols = []
            for h in range(HH):
                qh = q[:, h * DH:(h + 1) * DH]
                kh = k_ref[:, col0 + h * DH: col0 + (h + 1) * DH]
                s = lax.dot_general(
                    qh, kh, (((1,), (1,)), ((), ())),
                    preferred_element_type=jnp.float32,
                ) + bias
                e = jnp.exp(s)
                rs = 1.0 / jnp.sum(e, axis=1, keepdims=True)
                vh = v_ref[:, col0 + h * DH: col0 + (h + 1) * DH]
                ch = jnp.dot(e.astype(jnp.bfloat16), vh,
                             preferred_element_type=jnp.float32) * rs
                ctx_cols.append(ch.astype(jnp.bfloat16))
            ctx = jnp.concatenate(ctx_cols, axis=1)
            return lax.dot_general(
                ctx, woT, (((1,), (1,)), ((), ())),
                preferred_element_type=jnp.float32,
            )

        def hop(ring_slots, send_sems, recv_sems, h, src0, dst_dev):
            src = src0 if h == 0 else ring_slots.at[h - 1]
            return pltpu.make_async_remote_copy(
                src_ref=src,
                dst_ref=ring_slots.at[h],
                send_sem=send_sems.at[h],
                recv_sem=recv_sems.at[h],
                device_id=(dst_dev,),
                device_id_type=pl.DeviceIdType.MESH,
            )

        r0 = hop(slotr, sr_send, sr_recv, 0, wr_ref, right)
        l0 = hop(slotl, sl_send, sl_recv, 0, wl_ref, left)
        r0.start()
        l0.start()
        out_ref[...] = compute_half(wr_ref[...], 0)
        out_ref[...] += compute_half(wl_ref[...], N_DEV * DHALF)
        rprev, lprev = r0, l0
        for h in range(1, N_DEV):
            rprev.wait()
            if h < N_DEV - 1:
                rnext = hop(slotr, sr_send, sr_recv, h, None, right)
                rnext.start()
                rprev = rnext
            out_ref[...] += compute_half(slotr[h - 1], h * DHALF)
            lprev.wait()
            if h < N_DEV - 1:
                lnext = hop(slotl, sl_send, sl_recv, h, None, left)
                lnext.start()
                lprev = lnext
            out_ref[...] += compute_half(slotl[h - 1], (N_DEV + h) * DHALF)

    out = pl.pallas_call(
        body,
        out_shape=jax.ShapeDtypeStruct((S, D), jnp.float32),
        in_specs=[pl.BlockSpec(memory_space=pltpu.VMEM)] * 5,
        out_specs=pl.BlockSpec(memory_space=pltpu.VMEM),
        scratch_shapes=[
            pltpu.VMEM((N_DEV - 1, 2 * D, DHALF), jnp.bfloat16),
            pltpu.VMEM((N_DEV - 1, 2 * D, DHALF), jnp.bfloat16),
            pltpu.SemaphoreType.DMA((N_DEV - 1,)),
            pltpu.SemaphoreType.DMA((N_DEV - 1,)),
            pltpu.SemaphoreType.DMA((N_DEV - 1,)),
            pltpu.SemaphoreType.DMA((N_DEV - 1,)),
        ],
        compiler_params=pltpu.CompilerParams(
            collective_id=0, vmem_limit_bytes=56 * 1024 * 1024
        ),
    )(xb, wR, wL, Kb, Vb)
    return out.reshape(1, S, D)


# device time: 179992 ns/iter; 1.4045x vs baseline; 1.1425x over previous
import jax
import jax.numpy as jnp
from jax import lax
from jax.experimental import pallas as pl
from jax.experimental.pallas import tpu as pltpu

N_DEV = 4
S = 1024
D = 1024
HH = 4
DH = 128
DHALF = HH * DH
SCALE = 0.08838834764831843
NEG = -1e9


def kernel(x, Wq, K_ext, V_ext, Wo):
    my = lax.axis_index("i")

    xb = x.reshape(S, D).astype(jnp.bfloat16)
    Wq16 = Wq.astype(jnp.bfloat16)
    Wo16 = Wo.astype(jnp.bfloat16)

    order_r = (my - jnp.arange(N_DEV)) % N_DEV
    Km = lax.dynamic_index_in_dim(K_ext, my, 0, keepdims=False)
    Vm = lax.dynamic_index_in_dim(V_ext, my, 0, keepdims=False)
    Kb = jnp.take(Km.reshape(S, N_DEV, D), order_r, axis=1) \
        .reshape(S, N_DEV * D).astype(jnp.bfloat16)
    Vb = jnp.take(Vm.reshape(S, N_DEV, D), order_r, axis=1) \
        .reshape(S, N_DEV * D).astype(jnp.bfloat16)

    def body(x_ref, wq_ref, wo_ref, k_ref, v_ref, out_ref,
             sqr, sor, sql, sol,
             qr_s, qr_r, or_s, or_r, ql_s, ql_r, ol_s, ol_r):
        my_pos = lax.axis_index("i")
        right = lax.rem(my_pos + 1, N_DEV)
        left = lax.rem(my_pos + N_DEV - 1, N_DEV)

        barrier = pltpu.get_barrier_semaphore()
        for nbr in (left, right):
            pl.semaphore_signal(
                barrier, inc=1, device_id=(nbr,),
                device_id_type=pl.DeviceIdType.MESH,
            )
        pl.semaphore_wait(barrier, 2)

        qb = lax.broadcasted_iota(jnp.int32, (S, S), 0) // 64
        kb = lax.broadcasted_iota(jnp.int32, (S, S), 1) // 64
        mask = (qb == kb) | (kb == 0) | ((qb + kb) % 3 == 0)
        bias = jnp.where(mask, 0.0, NEG).astype(jnp.float32)

        xv = x_ref[...]

        def compute_half(wq_h, wo_h, col0):
            q = (jnp.dot(xv, wq_h, preferred_element_type=jnp.float32)
                 * SCALE).astype(jnp.bfloat16)
            ctx_cols = []
            for h in range(HH):
                qh = q[:, h * DH:(h + 1) * DH]
                kh = k_ref[:, col0 + h * DH: col0 + (h + 1) * DH]
                s = lax.dot_general(
                    qh, kh, (((1,), (1,)), ((), ())),
                    preferred_element_type=jnp.float32,
                ) + bias
                e = jnp.exp(s)
                rs = 1.0 / jnp.sum(e, axis=1, keepdims=True)
                vh = v_ref[:, col0 + h * DH: col0 + (h + 1) * DH]
                ch = jnp.dot(e.astype(jnp.bfloat16), vh,
                             preferred_element_type=jnp.float32) * rs
                ctx_cols.append(ch.astype(jnp.bfloat16))
            ctx = jnp.concatenate(ctx_cols, axis=1)
            return jnp.dot(ctx, wo_h, preferred_element_type=jnp.float32)

        def mk(slots, ssem, rsem, h, src, dev):
            return pltpu.make_async_remote_copy(
                src_ref=src, dst_ref=slots.at[h], send_sem=ssem.at[h],
                recv_sem=rsem.at[h], device_id=(dev,),
                device_id_type=pl.DeviceIdType.MESH,
            )

        rq = mk(sqr, qr_s, qr_r, 0, wq_ref.at[:, 0:DHALF], right)
        ro = mk(sor, or_s, or_r, 0, wo_ref.at[0:DHALF, :], right)
        lq = mk(sql, ql_s, ql_r, 0, wq_ref.at[:, DHALF:D], left)
        lo = mk(sol, ol_s, ol_r, 0, wo_ref.at[DHALF:D, :], left)
        for dma in (rq, ro, lq, lo):
            dma.start()
        out_ref[...] = compute_half(wq_ref[:, 0:DHALF], wo_ref[0:DHALF, :], 0)
        out_ref[...] += compute_half(wq_ref[:, DHALF:D], wo_ref[DHALF:D, :],
                                     DHALF)
        for h in range(1, N_DEV):
            rq.wait()
            ro.wait()
            if h < N_DEV - 1:
                rq = mk(sqr, qr_s, qr_r, h, sqr.at[h - 1], right)
                ro = mk(sor, or_s, or_r, h, sor.at[h - 1], right)
                rq.start()
                ro.start()
            out_ref[...] += compute_half(sqr[h - 1], sor[h - 1], h * D)
            lq.wait()
            lo.wait()
            if h < N_DEV - 1:
                lq = mk(sql, ql_s, ql_r, h, sql.at[h - 1], left)
                lo = mk(sol, ol_s, ol_r, h, sol.at[h - 1], left)
                lq.start()
                lo.start()
            out_ref[...] += compute_half(
                sql[h - 1], sol[h - 1], ((N_DEV - h) % N_DEV) * D + DHALF
            )

    nh = N_DEV - 1
    out = pl.pallas_call(
        body,
        out_shape=jax.ShapeDtypeStruct((S, D), jnp.float32),
        in_specs=[pl.BlockSpec(memory_space=pltpu.VMEM)] * 5,
        out_specs=pl.BlockSpec(memory_space=pltpu.VMEM),
        scratch_shapes=[
            pltpu.VMEM((nh, D, DHALF), jnp.bfloat16),
            pltpu.VMEM((nh, DHALF, D), jnp.bfloat16),
            pltpu.VMEM((nh, D, DHALF), jnp.bfloat16),
            pltpu.VMEM((nh, DHALF, D), jnp.bfloat16),
        ] + [pltpu.SemaphoreType.DMA((nh,))] * 8,
        compiler_params=pltpu.CompilerParams(
            collective_id=0, vmem_limit_bytes=56 * 1024 * 1024
        ),
    )(xb, Wq16, Wo16, Kb, Vb)
    return out.reshape(1, S, D)


# device time: 115077 ns/iter; 2.1968x vs baseline; 1.5641x over previous
import jax
import jax.numpy as jnp
from jax import lax
from jax.experimental import pallas as pl
from jax.experimental.pallas import tpu as pltpu

N_DEV = 4
S = 1024
D = 1024
HQ = 8
HH = 4
DH = 128
DHALF = HH * DH
SCALE = 0.08838834764831843
NEG = -1e9


def kernel(x, Wq, K_ext, V_ext, Wo):
    xb = x.reshape(S, D).astype(jnp.bfloat16)
    Wq16 = Wq.astype(jnp.bfloat16)
    Wo16 = Wo.astype(jnp.bfloat16)

    def body(x_ref, wq_ref, wo_ref, k_hbm, v_hbm, out_ref,
             sqr, sor, sql, sol, kst, vst,
             qr_s, qr_r, or_s, or_r, ql_s, ql_r, ol_s, ol_r, ksem, vsem):
        my_pos = lax.axis_index("i")
        right = lax.rem(my_pos + 1, N_DEV)
        left = lax.rem(my_pos + N_DEV - 1, N_DEV)

        def head_off(s):
            t = s // 2
            if s % 2 == 0:
                o = lax.rem(my_pos - t + N_DEV, N_DEV)
                return o * HQ
            o = lax.rem(my_pos + t, N_DEV)
            return o * HQ + HH

        fetches = {}

        def start_fetch(s):
            off = head_off(s)
            slot = s % 2
            kd = pltpu.make_async_copy(
                k_hbm.at[my_pos, :, pl.ds(off, HH), :], kst.at[slot],
                ksem.at[slot])
            vd = pltpu.make_async_copy(
                v_hbm.at[my_pos, :, pl.ds(off, HH), :], vst.at[slot],
                vsem.at[slot])
            kd.start()
            vd.start()
            fetches[s] = (kd, vd)

        start_fetch(0)
        start_fetch(1)

        barrier = pltpu.get_barrier_semaphore()
        for nbr in (left, right):
            pl.semaphore_signal(
                barrier, inc=1, device_id=(nbr,),
                device_id_type=pl.DeviceIdType.MESH,
            )
        pl.semaphore_wait(barrier, 2)

        qb = lax.broadcasted_iota(jnp.int32, (S, S), 0) // 64
        kb = lax.broadcasted_iota(jnp.int32, (S, S), 1) // 64
        mask = (qb == kb) | (kb == 0) | ((qb + kb) % 3 == 0)
        bias = jnp.where(mask, 0.0, NEG).astype(jnp.float32)

        xv = x_ref[...]

        def compute_half(s, wq_h, wo_h):
            slot = s % 2
            kd, vd = fetches[s]
            kd.wait()
            vd.wait()
            kv = kst[slot].reshape(S, DHALF).astype(jnp.bfloat16)
            vv = vst[slot].reshape(S, DHALF).astype(jnp.bfloat16)
            if s + 2 < 2 * N_DEV:
                start_fetch(s + 2)
            q = (jnp.dot(xv, wq_h, preferred_element_type=jnp.float32)
                 * SCALE).astype(jnp.bfloat16)
            ctx_cols = []
            for h in range(HH):
                qh = q[:, h * DH:(h + 1) * DH]
                kh = kv[:, h * DH:(h + 1) * DH]
                sc = lax.dot_general(
                    qh, kh, (((1,), (1,)), ((), ())),
                    preferred_element_type=jnp.float32,
                ) + bias
                e = jnp.exp(sc)
                rs = 1.0 / jnp.sum(e, axis=1, keepdims=True)
                vh = vv[:, h * DH:(h + 1) * DH]
                ch = jnp.dot(e.astype(jnp.bfloat16), vh,
                             preferred_element_type=jnp.float32) * rs
                ctx_cols.append(ch.astype(jnp.bfloat16))
            ctx = jnp.concatenate(ctx_cols, axis=1)
            return jnp.dot(ctx, wo_h, preferred_element_type=jnp.float32)

        def mk(slots, ssem, rsem, h, src, dev):
            return pltpu.make_async_remote_copy(
                src_ref=src, dst_ref=slots.at[h], send_sem=ssem.at[h],
                recv_sem=rsem.at[h], device_id=(dev,),
                device_id_type=pl.DeviceIdType.MESH,
            )

        rq = mk(sqr, qr_s, qr_r, 0, wq_ref.at[:, 0:DHALF], right)
        ro = mk(sor, or_s, or_r, 0, wo_ref.at[0:DHALF, :], right)
        lq = mk(sql, ql_s, ql_r, 0, wq_ref.at[:, DHALF:D], left)
        lo = mk(sol, ol_s, ol_r, 0, wo_ref.at[DHALF:D, :], left)
        for dma in (rq, ro, lq, lo):
            dma.start()
        out_ref[...] = compute_half(0, wq_ref[:, 0:DHALF], wo_ref[0:DHALF, :])
        out_ref[...] += compute_half(1, wq_ref[:, DHALF:D], wo_ref[DHALF:D, :])
        for h in range(1, N_DEV):
            rq.wait()
            ro.wait()
            if h < N_DEV - 1:
                rq = mk(sqr, qr_s, qr_r, h, sqr.at[h - 1], right)
                ro = mk(sor, or_s, or_r, h, sor.at[h - 1], right)
                rq.start()
                ro.start()
            out_ref[...] += compute_half(2 * h, sqr[h - 1], sor[h - 1])
            lq.wait()
            lo.wait()
            if h < N_DEV - 1:
                lq = mk(sql, ql_s, ql_r, h, sql.at[h - 1], left)
                lo = mk(sol, ol_s, ol_r, h, sol.at[h - 1], left)
                lq.start()
                lo.start()
            out_ref[...] += compute_half(2 * h + 1, sql[h - 1], sol[h - 1])

    nh = N_DEV - 1
    out = pl.pallas_call(
        body,
        out_shape=jax.ShapeDtypeStruct((S, D), jnp.float32),
        in_specs=[pl.BlockSpec(memory_space=pltpu.VMEM)] * 3
        + [pl.BlockSpec(memory_space=pl.ANY)] * 2,
        out_specs=pl.BlockSpec(memory_space=pltpu.VMEM),
        scratch_shapes=[
            pltpu.VMEM((nh, D, DHALF), jnp.bfloat16),
            pltpu.VMEM((nh, DHALF, D), jnp.bfloat16),
            pltpu.VMEM((nh, D, DHALF), jnp.bfloat16),
            pltpu.VMEM((nh, DHALF, D), jnp.bfloat16),
            pltpu.VMEM((2, S, HH, DH), jnp.float32),
            pltpu.VMEM((2, S, HH, DH), jnp.float32),
        ] + [pltpu.SemaphoreType.DMA((nh,))] * 8
        + [pltpu.SemaphoreType.DMA((2,))] * 2,
        compiler_params=pltpu.CompilerParams(
            collective_id=0, vmem_limit_bytes=56 * 1024 * 1024
        ),
    )(xb, Wq16, Wo16, K_ext, V_ext)
    return out.reshape(1, S, D)
